# baseline (speedup 1.0000x reference)
import functools

import jax
import jax.numpy as jnp
from jax import lax
from jax.experimental import pallas as pl
from jax.experimental.pallas import tpu as pltpu
from jax.experimental.pallas import tpu_sc as plsc

_VOCAB = 1000000
_DIM = 64
_BATCH = 16384
_CTX = 5

_NC = 2
_NS = 16
_NW = _NC * _NS
_BPW = _BATCH // _NW
_CHUNK = 128
_NCHUNK = _BPW // _CHUNK
_CROWS = _CHUNK * _CTX


def _skipgram_body(tgt_hbm, ctx_hbm, ttab_hbm, ctab_hbm, out_hbm,
                   tidx_v, cidx_v, we_v, ce_v, out_v, sem):
    wid = lax.axis_index("s") * _NC + lax.axis_index("c")
    base = wid * _BPW

    def chunk_body(ci, carry):
        b0 = base + ci * _CHUNK
        pltpu.sync_copy(tgt_hbm.at[pl.ds(b0, _CHUNK)], tidx_v)
        pltpu.sync_copy(ctx_hbm.at[pl.ds(b0 * _CTX, _CROWS)], cidx_v)

        cps = [pltpu.async_copy(ttab_hbm.at[tidx_v], we_v, sem)]
        for j in range(_CTX):
            cps.append(pltpu.async_copy(
                ctab_hbm.at[cidx_v.at[pl.ds(j * _CHUNK, _CHUNK)]],
                ce_v.at[pl.ds(j * _CHUNK, _CHUNK)], sem))
        for cp in cps:
            cp.wait()

        lanes = lax.iota(jnp.int32, 16)
        lane0 = lanes == 0
        perms = [lanes ^ s for s in (8, 4, 2, 1)]

        def _shuf(x, perm):
            dims = lax.GatherDimensionNumbers(
                offset_dims=(), collapsed_slice_dims=(0,),
                start_index_map=(0,))
            return lax.gather(x, perm[:, None], dims, (1,),
                              mode=lax.GatherScatterMode.PROMISE_IN_BOUNDS)

        def b_body(b, c2):
            we = [we_v[b, pl.ds(16 * j, 16)] for j in range(4)]
            for c in range(_CTX):
                p = b * _CTX + c
                acc = ce_v[p, pl.ds(0, 16)] * we[0]
                for j in range(1, 4):
                    acc = acc + ce_v[p, pl.ds(16 * j, 16)] * we[j]
                for perm in perms:
                    acc = acc + _shuf(acc, perm)
                plsc.store_scatter(out_v, [jnp.full((16,), p, jnp.int32)],
                                   acc, mask=lane0)
            return c2

        lax.fori_loop(0, _CHUNK, b_body, 0)
        pltpu.sync_copy(out_v, out_hbm.at[pl.ds(b0 * _CTX, _CROWS)])
        return carry

    lax.fori_loop(0, _NCHUNK, chunk_body, 0)


@jax.jit
def _skipgram(tgt, ctx, ttab, ctab):
    mesh = plsc.VectorSubcoreMesh(core_axis_name="c", subcore_axis_name="s")
    f = functools.partial(
        pl.kernel,
        out_type=jax.ShapeDtypeStruct((_BATCH * _CTX,), jnp.float32),
        mesh=mesh,
        scratch_types=[
            pltpu.VMEM((_CHUNK,), jnp.int32),
            pltpu.VMEM((_CROWS,), jnp.int32),
            pltpu.VMEM((_CHUNK, _DIM), jnp.float32),
            pltpu.VMEM((_CROWS, _DIM), jnp.float32),
            pltpu.VMEM((_CROWS,), jnp.float32),
            pltpu.SemaphoreType.DMA,
        ],
        compiler_params=pltpu.CompilerParams(needs_layout_passes=False,
                                             use_tc_tiling_on_sc=False),
    )(_skipgram_body)
    return f(tgt, ctx, ttab, ctab).reshape(_BATCH, _CTX)


def kernel(target, context, target_table, context_table):
    tgt = jnp.asarray(target, jnp.int32).reshape(_BATCH)
    ctx = jnp.asarray(context, jnp.int32).reshape(_BATCH * _CTX)
    return _skipgram(tgt, ctx, target_table, context_table)
